# baseline (speedup 1.0000x reference)
import jax
import jax.numpy as jnp
from jax.experimental import pallas as pl


def _relu_body(x_ref, o_ref):
    o_ref[...] = jnp.maximum(x_ref[...], 0.0)


def _pallas_relu(x):
    n, d = x.shape
    blk = 1024 if n % 1024 == 0 else 8
    while n % blk:
        blk //= 2
    return pl.pallas_call(
        _relu_body,
        out_shape=jax.ShapeDtypeStruct((n, d), x.dtype),
        grid=(n // blk,),
        in_specs=[pl.BlockSpec((blk, d), lambda i: (i, 0))],
        out_specs=pl.BlockSpec((blk, d), lambda i: (i, 0)),
    )(x)


def _gated_gcn(src, dst, n_nodes, h, e, W, b):
    Ah = h @ W[0] + b[0]
    Bh = h @ W[1] + b[1]
    Dh = h @ W[2] + b[2]
    Eh = h @ W[3] + b[3]
    Ce = e @ W[4] + b[4]
    e_hat = Dh[src] + Eh[dst] + Ce
    sigma = jax.nn.sigmoid(e_hat)
    num = jax.ops.segment_sum(sigma * Bh[src], dst, num_segments=n_nodes)
    den = jax.ops.segment_sum(sigma, dst, num_segments=n_nodes) + 1e-6
    h_new = _pallas_relu(Ah + num / den)
    e_new = _pallas_relu(e_hat)
    return h_new, e_new


def kernel(h, e, l, edge_index, lg_edge_index, W1, b1, W2, b2):
    h_out, m = _gated_gcn(edge_index[0], edge_index[1], h.shape[0], h, e, W1, b1)
    e_out, l_out = _gated_gcn(lg_edge_index[0], lg_edge_index[1], m.shape[0], m, l, W2, b2)
    return (h_out, e_out, l_out)


# speedup vs baseline: 11.1553x; 11.1553x over previous
import functools

import jax
import jax.numpy as jnp
from jax import lax
from jax.experimental import pallas as pl
from jax.experimental.pallas import tpu as pltpu
from jax.experimental.pallas import tpu_sc as plsc

D = 128
NW = 32


def _sc_gather_pair(db, eh, src, dst, ch):
    E = src.shape[0]
    bpw = E // NW
    nch = bpw // ch
    mesh = plsc.VectorSubcoreMesh(core_axis_name="c", subcore_axis_name="s")

    @functools.partial(
        pl.kernel,
        out_type=[
            jax.ShapeDtypeStruct((E, 2 * D), jnp.float32),
            jax.ShapeDtypeStruct((E, D), jnp.float32),
        ],
        mesh=mesh,
        scratch_types=[
            pltpu.VMEM((ch,), jnp.int32),
            pltpu.VMEM((ch,), jnp.int32),
            pltpu.VMEM((ch, 2 * D), jnp.float32),
            pltpu.VMEM((ch, D), jnp.float32),
            pltpu.SemaphoreType.DMA,
            pltpu.SemaphoreType.DMA,
        ],
    )
    def k(db_hbm, eh_hbm, src_hbm, dst_hbm, gdb_hbm, geh_hbm,
          si_v, di_v, dbuf, ebuf, sem1, sem2):
        w = lax.axis_index("s") * 2 + lax.axis_index("c")
        base = w * bpw

        def body(j, carry):
            off = base + j * ch
            pltpu.sync_copy(src_hbm.at[pl.ds(off, ch)], si_v)
            pltpu.sync_copy(dst_hbm.at[pl.ds(off, ch)], di_v)
            cp1 = pltpu.async_copy(db_hbm.at[si_v], dbuf, sem1)
            cp2 = pltpu.async_copy(eh_hbm.at[di_v], ebuf, sem2)
            cp1.wait()
            cp2.wait()
            pltpu.sync_copy(dbuf, gdb_hbm.at[pl.ds(off, ch)])
            pltpu.sync_copy(ebuf, geh_hbm.at[pl.ds(off, ch)])
            return carry

        lax.fori_loop(0, nch, body, 0)

    return k(db, eh, src, dst)


def _node_linear_body(x_ref, w_ref, b_ref, a_ref, db_ref, eh_ref):
    out = jnp.dot(x_ref[...], w_ref[...], preferred_element_type=jnp.float32)
    out = out + b_ref[...]
    a_ref[...] = out[:, :D]
    db_ref[...] = out[:, D:3 * D]
    eh_ref[...] = out[:, 3 * D:]


def _node_linear(x, W, b, blk):
    M = x.shape[0]
    Wp = jnp.concatenate([W[0], W[2], W[1], W[3]], axis=1)
    bp = jnp.concatenate([b[0], b[2], b[1], b[3]], axis=0)[None, :]
    return pl.pallas_call(
        _node_linear_body,
        grid=(M // blk,),
        in_specs=[
            pl.BlockSpec((blk, D), lambda i: (i, 0)),
            pl.BlockSpec((D, 4 * D), lambda i: (0, 0)),
            pl.BlockSpec((1, 4 * D), lambda i: (0, 0)),
        ],
        out_specs=[
            pl.BlockSpec((blk, D), lambda i: (i, 0)),
            pl.BlockSpec((blk, 2 * D), lambda i: (i, 0)),
            pl.BlockSpec((blk, D), lambda i: (i, 0)),
        ],
        out_shape=[
            jax.ShapeDtypeStruct((M, D), jnp.float32),
            jax.ShapeDtypeStruct((M, 2 * D), jnp.float32),
            jax.ShapeDtypeStruct((M, D), jnp.float32),
        ],
    )(x, Wp, bp)


def _edge_linear_body(x_ref, w_ref, b_ref, o_ref):
    o_ref[...] = (
        jnp.dot(x_ref[...], w_ref[...], preferred_element_type=jnp.float32)
        + b_ref[...]
    )


def _edge_linear(x, W, b, blk):
    M = x.shape[0]
    return pl.pallas_call(
        _edge_linear_body,
        grid=(M // blk,),
        in_specs=[
            pl.BlockSpec((blk, D), lambda i: (i, 0)),
            pl.BlockSpec((D, D), lambda i: (0, 0)),
            pl.BlockSpec((1, D), lambda i: (0, 0)),
        ],
        out_specs=pl.BlockSpec((blk, D), lambda i: (i, 0)),
        out_shape=jax.ShapeDtypeStruct((M, D), jnp.float32),
    )(x, W, b[None, :])


def _edge_elem_body(gdb_ref, geh_ref, ce_ref, pay_ref, m_ref):
    gdb = gdb_ref[...]
    e_hat = gdb[:, :D] + geh_ref[...] + ce_ref[...]
    sigma = jax.nn.sigmoid(e_hat)
    prod = sigma * gdb[:, D:]
    pay_ref[...] = jnp.concatenate([prod, sigma], axis=1)
    m_ref[...] = jnp.maximum(e_hat, 0.0)


def _edge_elem(gdb, geh, ce, blk):
    E = gdb.shape[0]
    return pl.pallas_call(
        _edge_elem_body,
        grid=(E // blk,),
        in_specs=[
            pl.BlockSpec((blk, 2 * D), lambda i: (i, 0)),
            pl.BlockSpec((blk, D), lambda i: (i, 0)),
            pl.BlockSpec((blk, D), lambda i: (i, 0)),
        ],
        out_specs=[
            pl.BlockSpec((blk, 2 * D), lambda i: (i, 0)),
            pl.BlockSpec((blk, D), lambda i: (i, 0)),
        ],
        out_shape=[
            jax.ShapeDtypeStruct((E, 2 * D), jnp.float32),
            jax.ShapeDtypeStruct((E, D), jnp.float32),
        ],
    )(gdb, geh, ce)


def _node_final_body(a_ref, acc_ref, o_ref):
    acc = acc_ref[...]
    o_ref[...] = jnp.maximum(
        a_ref[...] + acc[:, :D] / (acc[:, D:] + 1e-6), 0.0
    )


def _node_final(ah, acc, blk):
    M = ah.shape[0]
    return pl.pallas_call(
        _node_final_body,
        grid=(M // blk,),
        in_specs=[
            pl.BlockSpec((blk, D), lambda i: (i, 0)),
            pl.BlockSpec((blk, 2 * D), lambda i: (i, 0)),
        ],
        out_specs=pl.BlockSpec((blk, D), lambda i: (i, 0)),
        out_shape=jax.ShapeDtypeStruct((M, D), jnp.float32),
    )(ah, acc)


def _layer(x_nodes, x_edges, src, dst, W, b, node_blk, edge_blk):
    n_nodes = x_nodes.shape[0]
    ah, db, eh = _node_linear(x_nodes, W, b, node_blk)
    ce = _edge_linear(x_edges, W[4], b[4], edge_blk)
    gdb, geh = _sc_gather_pair(db, eh, src, dst, ch=200)
    payload, e_new = _edge_elem(gdb, geh, ce, edge_blk)
    acc = jax.ops.segment_sum(payload, dst, num_segments=n_nodes)
    h_new = _node_final(ah, acc, node_blk)
    return h_new, e_new


def kernel(h, e, l, edge_index, lg_edge_index, W1, b1, W2, b2):
    h_out, m = _layer(h, e, edge_index[0], edge_index[1], W1, b1,
                      node_blk=400, edge_blk=1600)
    e_out, l_out = _layer(m, l, lg_edge_index[0], lg_edge_index[1], W2, b2,
                          node_blk=1600, edge_blk=1600)
    return (h_out, e_out, l_out)
